# Initial kernel scaffold; baseline (speedup 1.0000x reference)
import jax
import jax.numpy as jnp
from jax.experimental import pallas as pl


def kernel(x, actions, tar_scores, geo, wall, category, batch, edge_index, params):
    raise NotImplementedError("write your pallas kernel here")


# speedup vs baseline: 2.9412x; 2.9412x over previous
import functools

import jax
import jax.numpy as jnp
from jax import lax
from jax.experimental import pallas as pl
from jax.experimental.pallas import tpu as pltpu
from jax.experimental.pallas import tpu_sc as plsc

N = 10000
E = 320000
NG = 256
HID = 64
EMB = 32
COND = 3 * EMB
NW = 32
NPT = 320
MAGIC = 13108
EPN = E + NW * 128
NB = 1000
EROW = E + 8
STAG = 4640
TRASH = 4624
NEG_INF = float("-inf")


def _wid():
    return lax.axis_index("s") * 2 + lax.axis_index("c")


def _iota16():
    return lax.broadcasted_iota(jnp.int32, (16,), 0)


def _tile_cnt_off(cvm, t):
    cnt = jnp.int32(0)
    off = jnp.int32(0)
    for g in range(2):
        gt = _iota16() + 16 * g
        cg = plsc.load_gather(cvm, [gt * 8])
        r128 = jnp.bitwise_and(cg + 127, jnp.int32(-128))
        off = off + jnp.sum(jnp.where(gt < t, r128, 0))
        cnt = cnt + jnp.sum(jnp.where(gt == t, cg, 0))
    return cnt, off


def _sc_partition_body(src_e, dst_e, srcp, dstp, counts, sbuf, dbuf, stag, dtag, cbuf):
    t = _wid()
    zero16 = jnp.zeros((16,), jnp.int32)

    def zinit(i, _):
        stag[pl.ds(i * 16, 16)] = zero16
        dtag[pl.ds(i * 16, 16)] = zero16
        return 0

    lax.fori_loop(0, STAG // 16, zinit, 0)

    def chunk(ch, carry):
        cursor, written = carry
        pltpu.sync_copy(src_e.at[pl.ds(ch * 512, 512)], sbuf)
        pltpu.sync_copy(dst_e.at[pl.ds(ch * 512, 512)], dbuf)
        for v in range(32):
            s16 = sbuf[pl.ds(v * 16, 16)]
            d16 = dbuf[pl.ds(v * 16, 16)]
            own = jnp.right_shift(d16 * MAGIC, 22)
            m = own == t
            mi = m.astype(jnp.int32)
            rank = plsc.cumsum(mi) - mi
            dest = jnp.where(m, cursor + rank, TRASH + _iota16())
            plsc.store_scatter(stag, [dest], s16)
            plsc.store_scatter(dtag, [dest], d16)
            cursor = cursor + jnp.sum(mi)

        def flush(cur, wr):
            pltpu.sync_copy(stag.at[pl.ds(0, 4096)], srcp.at[pl.ds(pl.multiple_of(t * EROW + wr, 8), 4096)])
            pltpu.sync_copy(dtag.at[pl.ds(0, 4096)], dstp.at[pl.ds(pl.multiple_of(t * EROW + wr, 8), 4096)])
            for v in range(32):
                stag[pl.ds(v * 16, 16)] = stag[pl.ds(4096 + v * 16, 16)]
                dtag[pl.ds(v * 16, 16)] = dtag[pl.ds(4096 + v * 16, 16)]
            return cur - 4096, wr + 4096

        cursor, written = lax.cond(cursor >= 4096, flush,
                                   lambda c, w: (c, w), cursor, written)
        return cursor, written

    cursor, written = lax.fori_loop(0, E // 512, chunk, (jnp.int32(0), jnp.int32(0)))

    def tail(i, _):
        pltpu.sync_copy(stag.at[pl.ds(i * 8, 8)], srcp.at[pl.ds(pl.multiple_of(t * EROW + written + i * 8, 8), 8)])
        pltpu.sync_copy(dtag.at[pl.ds(i * 8, 8)], dstp.at[pl.ds(pl.multiple_of(t * EROW + written + i * 8, 8), 8)])
        return 0

    lax.fori_loop(0, jnp.right_shift(cursor + 7, 3), tail, 0)
    cbuf[...] = jnp.full((16,), written + cursor, jnp.int32)
    pltpu.sync_copy(cbuf.at[pl.ds(0, 8)], counts.at[pl.ds(pl.multiple_of(t * 8, 8), 8)])


def _sc_partition(src_e, dst_e):
    mesh = plsc.VectorSubcoreMesh(core_axis_name="c", subcore_axis_name="s")
    f = pl.kernel(
        _sc_partition_body,
        out_type=[
            jax.ShapeDtypeStruct((NW * EROW,), jnp.int32),
            jax.ShapeDtypeStruct((NW * EROW,), jnp.int32),
            jax.ShapeDtypeStruct((NW * 8,), jnp.int32),
        ],
        mesh=mesh,
        compiler_params=pltpu.CompilerParams(needs_layout_passes=False),
        scratch_types=[
            pltpu.VMEM((512,), jnp.int32),
            pltpu.VMEM((512,), jnp.int32),
            pltpu.VMEM((STAG,), jnp.int32),
            pltpu.VMEM((STAG,), jnp.int32),
            pltpu.VMEM((16,), jnp.int32),
        ],
    )
    return f(src_e, dst_e)


def _sc_gather_body(atab, btab, srcp, dstp, counts, ag, bg,
                    cvm, sloc, dloc, ia, ib, abuf, bbuf, sem_a, sem_b):
    t = _wid()
    pltpu.sync_copy(counts, cvm)
    cnt, off = _tile_cnt_off(cvm, t)
    nch = jnp.right_shift(cnt + 127, 7)

    def chunk(c, _):
        pltpu.sync_copy(srcp.at[pl.ds(pl.multiple_of(t * EROW + c * 128, 8), 128)], sloc)
        pltpu.sync_copy(dstp.at[pl.ds(pl.multiple_of(t * EROW + c * 128, 8), 128)], dloc)
        for v in range(8):
            lane = c * 128 + v * 16 + _iota16()
            ok = lane < cnt
            ia[pl.ds(v * 16, 16)] = jnp.where(ok, dloc[pl.ds(v * 16, 16)], 0)
            ib[pl.ds(v * 16, 16)] = jnp.where(ok, sloc[pl.ds(v * 16, 16)], 0)
        ca = pltpu.async_copy(atab.at[ia], abuf, sem_a)
        cb = pltpu.async_copy(btab.at[ib], bbuf, sem_b)
        ca.wait()
        cb.wait()
        pltpu.sync_copy(abuf, ag.at[pl.ds(pl.multiple_of(off + c * 128, 8), 128), :])
        pltpu.sync_copy(bbuf, bg.at[pl.ds(pl.multiple_of(off + c * 128, 8), 128), :])
        return 0

    lax.fori_loop(0, nch, chunk, 0)


def _sc_gather(atab, btab, srcp, dstp, counts):
    mesh = plsc.VectorSubcoreMesh(core_axis_name="c", subcore_axis_name="s")
    f = pl.kernel(
        _sc_gather_body,
        out_type=[
            jax.ShapeDtypeStruct((EPN, 2 * HID), jnp.float32),
            jax.ShapeDtypeStruct((EPN, 2 * HID), jnp.float32),
        ],
        mesh=mesh,
        compiler_params=pltpu.CompilerParams(needs_layout_passes=False),
        scratch_types=[
            pltpu.VMEM((NW * 8,), jnp.int32),
            pltpu.VMEM((128,), jnp.int32),
            pltpu.VMEM((128,), jnp.int32),
            pltpu.VMEM((128,), jnp.int32),
            pltpu.VMEM((128,), jnp.int32),
            pltpu.VMEM((128, 2 * HID), jnp.float32),
            pltpu.VMEM((128, 2 * HID), jnp.float32),
            pltpu.SemaphoreType.DMA,
            pltpu.SemaphoreType.DMA,
        ],
    )
    return f(atab, btab, srcp, dstp, counts)


def _sc_scatter_body(msg, dstp, counts, agg, cvm, dbuf, mbuf, acc):
    t = _wid()
    pltpu.sync_copy(counts, cvm)
    cnt, off = _tile_cnt_off(cvm, t)
    base = t * NPT
    ninf = jnp.full((16,), NEG_INF, jnp.float32)

    def zinit(r, _):
        for v in range(8):
            acc[r, pl.ds(v * 16, 16)] = ninf
        return 0

    lax.fori_loop(0, NPT, zinit, 0)

    nch = jnp.right_shift(cnt + 255, 8)

    def chunk(c, _):
        pltpu.sync_copy(msg.at[pl.ds(pl.multiple_of(off + c * 256, 8), 256), :], mbuf)
        pltpu.sync_copy(dstp.at[pl.ds(pl.multiple_of(t * EROW + c * 256, 8), 256)], dbuf)
        n_c = cnt - c * 256

        def vreg16(v, _):
            d16 = dbuf[pl.ds(v * 16, 16)] - base
            for j in range(16):
                d = jnp.sum(jnp.where(_iota16() == j, d16, 0))
                e = v * 16 + j

                @pl.when(e < n_c)
                def _():
                    for f in range(8):
                        mv = mbuf[e, pl.ds(f * 16, 16)]
                        av = acc[d, pl.ds(f * 16, 16)]
                        acc[d, pl.ds(f * 16, 16)] = jnp.maximum(av, mv)
            return 0

        lax.fori_loop(0, 16, vreg16, 0)
        return 0

    lax.fori_loop(0, nch, chunk, 0)
    pltpu.sync_copy(acc, agg.at[pl.ds(base, NPT), :])


def _sc_scatter(msg, dstp, counts):
    mesh = plsc.VectorSubcoreMesh(core_axis_name="c", subcore_axis_name="s")
    f = pl.kernel(
        _sc_scatter_body,
        out_type=jax.ShapeDtypeStruct((NW * NPT, 2 * HID), jnp.float32),
        mesh=mesh,
        compiler_params=pltpu.CompilerParams(needs_layout_passes=False),
        scratch_types=[
            pltpu.VMEM((NW * 8,), jnp.int32),
            pltpu.VMEM((256,), jnp.int32),
            pltpu.VMEM((256, 2 * HID), jnp.float32),
            pltpu.VMEM((NPT, 2 * HID), jnp.float32),
        ],
    )
    return f(msg, dstp, counts)


def _mm(x, w):
    return jnp.dot(x, w, preferred_element_type=jnp.float32)


def _tc_dense0_body(nin, cat, bat, wall,
                    wi1, bi1, wi2, bi2, ww1, bw1, ww2, bw2,
                    wg1, bg1, wg2, bg2, emt, wem, bem, wm1, bm1,
                    a_out, b_out, cond_out):
    x10 = nin[:, :10]
    geo = nin[:, 10:12]
    oh_c = (cat[...] == lax.broadcasted_iota(jnp.int32, (NB, 10), 1)).astype(jnp.float32)
    oh_b = (bat[...] == lax.broadcasted_iota(jnp.int32, (NB, NG), 1)).astype(jnp.float32)
    for b in range(2):
        h0 = jnp.tanh(_mm(jnp.tanh(_mm(x10, wi1[b]) + bi1[b]), wi2[b]) + bi2[b])
        ef = jnp.tanh(_mm(jnp.tanh(_mm(oh_c, emt[b])), wem[b]) + bem[b])
        wf_tab = _mm(jnp.tanh(_mm(wall[...], ww1[b]) + bw1[b]), ww2[b]) + bw2[b]
        wf = jnp.tanh(_mm(oh_b, wf_tab))
        gf = jnp.tanh(_mm(jnp.tanh(_mm(geo, wg1[b]) + bg1[b]), wg2[b]) + bg2[b])
        cond = jnp.concatenate([ef, wf, gf], axis=1)
        h = jnp.concatenate([h0, cond], axis=1)
        wa = wm1[b, :160, :] - wm1[b, 160:, :]
        a_out[:, b * HID:(b + 1) * HID] = _mm(h, wa) + bm1[b]
        b_out[:, b * HID:(b + 1) * HID] = _mm(h, wm1[b, 160:, :])
        cond_out[b] = cond


def _tc_dense0(nin, cat, bat, wall, p):
    grid = (N // NB,)
    bs_w = lambda shape: pl.BlockSpec((2,) + shape, lambda i: (0, 0, 0))
    f = pl.pallas_call(
        _tc_dense0_body,
        grid=grid,
        in_specs=[
            pl.BlockSpec((NB, 12), lambda i: (i, 0)),
            pl.BlockSpec((NB, 1), lambda i: (i, 0)),
            pl.BlockSpec((NB, 1), lambda i: (i, 0)),
            pl.BlockSpec((NG, 1), lambda i: (0, 0)),
            bs_w((10, HID)), bs_w((1, HID)), bs_w((HID, HID)), bs_w((1, HID)),
            bs_w((1, EMB)), bs_w((1, EMB)), bs_w((EMB, EMB)), bs_w((1, EMB)),
            bs_w((2, EMB)), bs_w((1, EMB)), bs_w((EMB, EMB)), bs_w((1, EMB)),
            bs_w((10, EMB)), bs_w((EMB, EMB)), bs_w((1, EMB)),
            bs_w((2 * (HID + COND), HID)), bs_w((1, HID)),
        ],
        out_specs=[
            pl.BlockSpec((NB, 2 * HID), lambda i: (i, 0)),
            pl.BlockSpec((NB, 2 * HID), lambda i: (i, 0)),
            pl.BlockSpec((2, NB, COND), lambda i: (0, i, 0)),
        ],
        out_shape=[
            jax.ShapeDtypeStruct((N, 2 * HID), jnp.float32),
            jax.ShapeDtypeStruct((N, 2 * HID), jnp.float32),
            jax.ShapeDtypeStruct((2, N, COND), jnp.float32),
        ],
    )
    return f(nin, cat, bat, wall,
             p["wi1"], p["bi1"], p["wi2"], p["bi2"],
             p["ww1"], p["bw1"], p["ww2"], p["bw2"],
             p["wg1"], p["bg1"], p["wg2"], p["bg2"],
             p["emt"], p["wem"], p["bem"], p["wm1"], p["bm1"])


def _tc_msg_body(ag, bg, w2, b2, out):
    z = jnp.tanh(ag[...] + bg[...])
    out[...] = _mm(z, w2[...]) + b2[...]


def _tc_msg(ag, bg, w2d, b2c):
    blk = 1024
    grid = (EPN // blk,)
    f = pl.pallas_call(
        _tc_msg_body,
        grid=grid,
        in_specs=[
            pl.BlockSpec((blk, 2 * HID), lambda i: (i, 0)),
            pl.BlockSpec((blk, 2 * HID), lambda i: (i, 0)),
            pl.BlockSpec((2 * HID, 2 * HID), lambda i: (0, 0)),
            pl.BlockSpec((1, 2 * HID), lambda i: (0, 0)),
        ],
        out_specs=pl.BlockSpec((blk, 2 * HID), lambda i: (i, 0)),
        out_shape=jax.ShapeDtypeStruct((EPN, 2 * HID), jnp.float32),
    )
    return f(ag, bg, w2d, b2c)


def _tc_mid_body(agg, cond, wm, bm, a_out, b_out):
    for b in range(2):
        a = agg[:, b * HID:(b + 1) * HID]
        h1 = jnp.tanh(jnp.where(a == NEG_INF, 0.0, a))
        h = jnp.concatenate([h1, cond[b]], axis=1)
        wa = wm[b, :160, :] - wm[b, 160:, :]
        a_out[:, b * HID:(b + 1) * HID] = _mm(h, wa) + bm[b]
        b_out[:, b * HID:(b + 1) * HID] = _mm(h, wm[b, 160:, :])


def _tc_mid(agg, cond, wm, bm):
    grid = (N // NB,)
    f = pl.pallas_call(
        _tc_mid_body,
        grid=grid,
        in_specs=[
            pl.BlockSpec((NB, 2 * HID), lambda i: (i, 0)),
            pl.BlockSpec((2, NB, COND), lambda i: (0, i, 0)),
            pl.BlockSpec((2, 2 * (HID + COND), HID), lambda i: (0, 0, 0)),
            pl.BlockSpec((2, 1, HID), lambda i: (0, 0, 0)),
        ],
        out_specs=[
            pl.BlockSpec((NB, 2 * HID), lambda i: (i, 0)),
            pl.BlockSpec((NB, 2 * HID), lambda i: (i, 0)),
        ],
        out_shape=[
            jax.ShapeDtypeStruct((N, 2 * HID), jnp.float32),
            jax.ShapeDtypeStruct((N, 2 * HID), jnp.float32),
        ],
    )
    return f(agg, cond, wm, bm)


def _tc_tail_body(agg, cond, wt1, bt1, wt2, bt2, out):
    for b in range(2):
        a = agg[:, b * HID:(b + 1) * HID]
        h2 = jnp.tanh(jnp.where(a == NEG_INF, 0.0, a))
        h = jnp.concatenate([h2, cond[b]], axis=1)
        out[b] = _mm(jnp.tanh(_mm(h, wt1[b]) + bt1[b]), wt2[b]) + bt2[b]


def _tc_tail(agg, cond, wt1, bt1, wt2, bt2):
    grid = (N // NB,)
    f = pl.pallas_call(
        _tc_tail_body,
        grid=grid,
        in_specs=[
            pl.BlockSpec((NB, 2 * HID), lambda i: (i, 0)),
            pl.BlockSpec((2, NB, COND), lambda i: (0, i, 0)),
            pl.BlockSpec((2, HID + COND, HID), lambda i: (0, 0, 0)),
            pl.BlockSpec((2, 1, HID), lambda i: (0, 0, 0)),
            pl.BlockSpec((2, HID, 8), lambda i: (0, 0, 0)),
            pl.BlockSpec((2, 1, 8), lambda i: (0, 0, 0)),
        ],
        out_specs=pl.BlockSpec((2, NB, 8), lambda i: (0, i, 0)),
        out_shape=jax.ShapeDtypeStruct((2, N, 8), jnp.float32),
    )
    return f(agg, cond, wt1, bt1, wt2, bt2)


def _stack_params(params):
    q = [params["q1"], params["q2"]]

    def st(path):
        def get(p):
            v = p
            for k in path:
                v = v[k]
            return v
        return jnp.stack([get(q[0]), get(q[1])])

    def lin2(v, r=None):
        return v[:, None, :]

    p = {
        "wi1": st(["init_enc", "l1", "W"]), "bi1": lin2(st(["init_enc", "l1", "b"])),
        "wi2": st(["init_enc", "l2", "W"]), "bi2": lin2(st(["init_enc", "l2", "b"])),
        "ww1": st(["wall_enc", "l1", "W"]), "bw1": lin2(st(["wall_enc", "l1", "b"])),
        "ww2": st(["wall_enc", "l2", "W"]), "bw2": lin2(st(["wall_enc", "l2", "b"])),
        "wg1": st(["geo_enc", "l1", "W"]), "bg1": lin2(st(["geo_enc", "l1", "b"])),
        "wg2": st(["geo_enc", "l2", "W"]), "bg2": lin2(st(["geo_enc", "l2", "b"])),
        "emt": st(["emb_table"]),
        "wem": st(["emb_lin", "W"]), "bem": lin2(st(["emb_lin", "b"])),
        "wm1": st(["mlp1", "l1", "W"]), "bm1": lin2(st(["mlp1", "l1", "b"])),
        "wm1b": st(["mlp1", "l2", "W"]), "bm1b": lin2(st(["mlp1", "l2", "b"])),
        "wm2": st(["mlp2", "l1", "W"]), "bm2": lin2(st(["mlp2", "l1", "b"])),
        "wm2b": st(["mlp2", "l2", "W"]), "bm2b": lin2(st(["mlp2", "l2", "b"])),
        "wt1": st(["tail", "l1", "W"]), "bt1": lin2(st(["tail", "l1", "b"])),
    }
    for i, nm in ((1, "mlp1"), (2, "mlp2")):
        w = st([nm, "l2", "W"])
        bb = st([nm, "l2", "b"])
        wd = jnp.zeros((2 * HID, 2 * HID), jnp.float32)
        wd = wd.at[:HID, :HID].set(w[0]).at[HID:, HID:].set(w[1])
        p[f"w2d{i}"] = wd
        p[f"b2c{i}"] = jnp.concatenate([bb[0], bb[1]])[None, :]

    wt2 = st(["tail", "l2", "W"])
    bt2 = lin2(st(["tail", "l2", "b"]))
    p["wt2"] = jnp.pad(wt2, ((0, 0), (0, 0), (0, 7)))
    p["bt2"] = jnp.pad(bt2, ((0, 0), (0, 0), (0, 7)))
    return p


def kernel(x, actions, tar_scores, geo, wall, category, batch, edge_index, params):
    cat = category.astype(jnp.int32)[:, None]
    bat = batch.astype(jnp.int32)[:, None]
    ei = edge_index.astype(jnp.int32)
    nin = jnp.concatenate([x, actions, tar_scores, geo], axis=1)
    p = _stack_params(params)

    srcp, dstp, counts = _sc_partition(ei[0], ei[1])

    a1, b1, cond = _tc_dense0(nin, cat, bat, wall, p)
    ag, bg = _sc_gather(a1, b1, srcp, dstp, counts)
    m1 = _tc_msg(ag, bg, p["w2d1"], p["b2c1"])
    agg1 = _sc_scatter(m1, dstp, counts)

    a2, b2 = _tc_mid(agg1[:N, :], cond, p["wm2"], p["bm2"])
    ag2, bg2 = _sc_gather(a2, b2, srcp, dstp, counts)
    m2 = _tc_msg(ag2, bg2, p["w2d2"], p["b2c2"])
    agg2 = _sc_scatter(m2, dstp, counts)

    q = _tc_tail(agg2[:N, :], cond, p["wt1"], p["bt1"], p["wt2"], p["bt2"])
    return (q[0, :, :1], q[1, :, :1])


# speedup vs baseline: 3.3602x; 1.1425x over previous
import functools

import jax
import jax.numpy as jnp
from jax import lax
from jax.experimental import pallas as pl
from jax.experimental.pallas import tpu as pltpu
from jax.experimental.pallas import tpu_sc as plsc

N = 10000
E = 320000
NG = 256
HID = 64
EMB = 32
COND = 3 * EMB
NW = 32
NPT = 320
MAGIC = 13108
EPN = 328704
NB = 1000
EROW = E + 8
STAG = 5408
TRASH = 5392
PCH = 1280
NPCH = E // PCH
NEG_INF = float("-inf")


def _wid():
    return lax.axis_index("s") * 2 + lax.axis_index("c")


def _iota16():
    return lax.broadcasted_iota(jnp.int32, (16,), 0)


def _tile_cnt_off(cvm, t):
    cnt = jnp.int32(0)
    off = jnp.int32(0)
    for g in range(2):
        gt = _iota16() + 16 * g
        cg = plsc.load_gather(cvm, [gt * 8])
        r256 = jnp.bitwise_and(cg + 255, jnp.int32(-256))
        off = off + jnp.sum(jnp.where(gt < t, r256, 0))
        cnt = cnt + jnp.sum(jnp.where(gt == t, cg, 0))
    return cnt, off


def _sc_partition_body(src_e, dst_e, srcp, dstp, counts,
                       sbuf, dbuf, stag, dtag, cbuf, ss0, ss1, sd0, sd1):
    t = _wid()
    zero16 = jnp.zeros((16,), jnp.int32)
    sems = ((ss0, sd0), (ss1, sd1))

    def zinit(i, _):
        stag[pl.ds(i * 16, 16)] = zero16
        dtag[pl.ds(i * 16, 16)] = zero16
        return 0

    lax.fori_loop(0, STAG // 16, zinit, 0)

    def issue(ch, p):
        cs = pltpu.async_copy(src_e.at[pl.ds(pl.multiple_of(ch * PCH, 8), PCH)],
                              sbuf.at[p], sems[p][0])
        cd = pltpu.async_copy(dst_e.at[pl.ds(pl.multiple_of(ch * PCH, 8), PCH)],
                              dbuf.at[p], sems[p][1])
        return cs, cd

    pre = issue(0, 0)

    def pair(i, carry):
        cursor, written = carry
        for p in range(2):
            ch = 2 * i + p

            @pl.when(ch + 1 < NPCH)
            def _():
                issue(ch + 1, 1 - p)

            pltpu.make_async_copy(src_e.at[pl.ds(0, PCH)], sbuf.at[p], sems[p][0]).wait()
            pltpu.make_async_copy(dst_e.at[pl.ds(0, PCH)], dbuf.at[p], sems[p][1]).wait()
            for v in range(PCH // 16):
                s16 = sbuf[p, pl.ds(v * 16, 16)]
                d16 = dbuf[p, pl.ds(v * 16, 16)]
                own = jnp.right_shift(d16 * MAGIC, 22)
                m = own == t
                mi = m.astype(jnp.int32)
                rank = plsc.cumsum(mi) - mi
                dest = jnp.where(m, cursor + rank, TRASH + _iota16())
                plsc.store_scatter(stag, [dest], s16)
                plsc.store_scatter(dtag, [dest], d16)
                cursor = cursor + jnp.sum(mi)

            def flush(cur, wr):
                pltpu.sync_copy(stag.at[pl.ds(0, 4096)],
                                srcp.at[pl.ds(pl.multiple_of(t * EROW + wr, 8), 4096)])
                pltpu.sync_copy(dtag.at[pl.ds(0, 4096)],
                                dstp.at[pl.ds(pl.multiple_of(t * EROW + wr, 8), 4096)])
                for v in range(82):
                    stag[pl.ds(v * 16, 16)] = stag[pl.ds(4096 + v * 16, 16)]
                    dtag[pl.ds(v * 16, 16)] = dtag[pl.ds(4096 + v * 16, 16)]
                return cur - 4096, wr + 4096

            cursor, written = lax.cond(cursor >= 4096, flush,
                                       lambda c, w: (c, w), cursor, written)
        return cursor, written

    cursor, written = lax.fori_loop(0, NPCH // 2, pair,
                                    (jnp.int32(0), jnp.int32(0)))

    def tail(i, _):
        pltpu.sync_copy(stag.at[pl.ds(i * 8, 8)],
                        srcp.at[pl.ds(pl.multiple_of(t * EROW + written + i * 8, 8), 8)])
        pltpu.sync_copy(dtag.at[pl.ds(i * 8, 8)],
                        dstp.at[pl.ds(pl.multiple_of(t * EROW + written + i * 8, 8), 8)])
        return 0

    lax.fori_loop(0, jnp.right_shift(cursor + 7, 3), tail, 0)
    cbuf[...] = jnp.full((16,), written + cursor, jnp.int32)
    pltpu.sync_copy(cbuf.at[pl.ds(0, 8)], counts.at[pl.ds(pl.multiple_of(t * 8, 8), 8)])


def _sc_partition(src_e, dst_e):
    mesh = plsc.VectorSubcoreMesh(core_axis_name="c", subcore_axis_name="s")
    f = pl.kernel(
        _sc_partition_body,
        out_type=[
            jax.ShapeDtypeStruct((NW * EROW,), jnp.int32),
            jax.ShapeDtypeStruct((NW * EROW,), jnp.int32),
            jax.ShapeDtypeStruct((NW * 8,), jnp.int32),
        ],
        mesh=mesh,
        compiler_params=pltpu.CompilerParams(needs_layout_passes=False),
        scratch_types=[
            pltpu.VMEM((2, PCH), jnp.int32),
            pltpu.VMEM((2, PCH), jnp.int32),
            pltpu.VMEM((STAG,), jnp.int32),
            pltpu.VMEM((STAG,), jnp.int32),
            pltpu.VMEM((16,), jnp.int32),
            pltpu.SemaphoreType.DMA,
            pltpu.SemaphoreType.DMA,
            pltpu.SemaphoreType.DMA,
            pltpu.SemaphoreType.DMA,
        ],
    )
    return f(src_e, dst_e)


def _sc_gather_body(atab, btab, srcp, dstp, counts, ag, bg,
                    cvm, sloc, dloc, ia, ia2, ib, ib2, abuf, bbuf,
                    sem_a, sem_a2, sem_b, sem_b2):
    t = _wid()
    pltpu.sync_copy(counts, cvm)
    cnt, off = _tile_cnt_off(cvm, t)
    nch = jnp.right_shift(cnt + 255, 8)

    def chunk(c, _):
        pltpu.sync_copy(srcp.at[pl.ds(pl.multiple_of(t * EROW + c * 256, 8), 256)], sloc)
        pltpu.sync_copy(dstp.at[pl.ds(pl.multiple_of(t * EROW + c * 256, 8), 256)], dloc)
        for v in range(16):
            lane = c * 256 + v * 16 + _iota16()
            ok = lane < cnt
            dv = jnp.where(ok, dloc[pl.ds(v * 16, 16)], 0)
            sv = jnp.where(ok, sloc[pl.ds(v * 16, 16)], 0)
            if v < 8:
                ia[pl.ds(v * 16, 16)] = dv
                ib[pl.ds(v * 16, 16)] = sv
            else:
                ia2[pl.ds((v - 8) * 16, 16)] = dv
                ib2[pl.ds((v - 8) * 16, 16)] = sv
        c1 = pltpu.async_copy(atab.at[ia], abuf.at[pl.ds(0, 128), :], sem_a)
        c2 = pltpu.async_copy(atab.at[ia2], abuf.at[pl.ds(128, 128), :], sem_a2)
        c3 = pltpu.async_copy(btab.at[ib], bbuf.at[pl.ds(0, 128), :], sem_b)
        c4 = pltpu.async_copy(btab.at[ib2], bbuf.at[pl.ds(128, 128), :], sem_b2)
        c1.wait()
        c2.wait()
        c3.wait()
        c4.wait()
        pltpu.sync_copy(abuf, ag.at[pl.ds(pl.multiple_of(off + c * 256, 8), 256), :])
        pltpu.sync_copy(bbuf, bg.at[pl.ds(pl.multiple_of(off + c * 256, 8), 256), :])
        return 0

    lax.fori_loop(0, nch, chunk, 0)


def _sc_gather(atab, btab, srcp, dstp, counts):
    mesh = plsc.VectorSubcoreMesh(core_axis_name="c", subcore_axis_name="s")
    f = pl.kernel(
        _sc_gather_body,
        out_type=[
            jax.ShapeDtypeStruct((EPN, 2 * HID), jnp.float32),
            jax.ShapeDtypeStruct((EPN, 2 * HID), jnp.float32),
        ],
        mesh=mesh,
        compiler_params=pltpu.CompilerParams(needs_layout_passes=False),
        scratch_types=[
            pltpu.VMEM((NW * 8,), jnp.int32),
            pltpu.VMEM((256,), jnp.int32),
            pltpu.VMEM((256,), jnp.int32),
            pltpu.VMEM((128,), jnp.int32),
            pltpu.VMEM((128,), jnp.int32),
            pltpu.VMEM((128,), jnp.int32),
            pltpu.VMEM((128,), jnp.int32),
            pltpu.VMEM((256, 2 * HID), jnp.float32),
            pltpu.VMEM((256, 2 * HID), jnp.float32),
            pltpu.SemaphoreType.DMA,
            pltpu.SemaphoreType.DMA,
            pltpu.SemaphoreType.DMA,
            pltpu.SemaphoreType.DMA,
        ],
    )
    return f(atab, btab, srcp, dstp, counts)


def _sc_scatter_body(msg, dstp, counts, agg, cvm, dbuf, mbuf, acc):
    t = _wid()
    pltpu.sync_copy(counts, cvm)
    cnt, off = _tile_cnt_off(cvm, t)
    base = t * NPT
    ninf = jnp.full((16,), NEG_INF, jnp.float32)

    def zinit(r, _):
        for v in range(8):
            acc[r, pl.ds(v * 16, 16)] = ninf
        return 0

    lax.fori_loop(0, NPT, zinit, 0)

    nch = jnp.right_shift(cnt + 255, 8)

    def chunk(c, _):
        pltpu.sync_copy(msg.at[pl.ds(pl.multiple_of(off + c * 256, 8), 256), :], mbuf)
        pltpu.sync_copy(dstp.at[pl.ds(pl.multiple_of(t * EROW + c * 256, 8), 256)], dbuf)
        n_c = cnt - c * 256

        nv = jnp.minimum(n_c, jnp.int32(256))
        nfull = jnp.right_shift(nv, 4)

        def vreg16(v, _):
            d16 = dbuf[pl.ds(v * 16, 16)] - base
            for j in range(16):
                d = jnp.sum(jnp.where(_iota16() == j, d16, 0))
                e = v * 16 + j
                for f in range(8):
                    mv = mbuf[e, pl.ds(f * 16, 16)]
                    av = acc[d, pl.ds(f * 16, 16)]
                    acc[d, pl.ds(f * 16, 16)] = jnp.maximum(av, mv)
            return 0

        lax.fori_loop(0, nfull, vreg16, 0)
        rem = jnp.bitwise_and(nv, 15)

        @pl.when(rem > 0)
        def _():
            v = nfull
            d16 = dbuf[pl.ds(v * 16, 16)] - base
            for j in range(16):
                d = jnp.sum(jnp.where(_iota16() == j, d16, 0))
                e = v * 16 + j

                @pl.when(j < rem)
                def _():
                    for f in range(8):
                        mv = mbuf[e, pl.ds(f * 16, 16)]
                        av = acc[d, pl.ds(f * 16, 16)]
                        acc[d, pl.ds(f * 16, 16)] = jnp.maximum(av, mv)
        return 0

    lax.fori_loop(0, nch, chunk, 0)
    pltpu.sync_copy(acc, agg.at[pl.ds(base, NPT), :])


def _sc_scatter(msg, dstp, counts):
    mesh = plsc.VectorSubcoreMesh(core_axis_name="c", subcore_axis_name="s")
    f = pl.kernel(
        _sc_scatter_body,
        out_type=jax.ShapeDtypeStruct((NW * NPT, 2 * HID), jnp.float32),
        mesh=mesh,
        compiler_params=pltpu.CompilerParams(needs_layout_passes=False),
        scratch_types=[
            pltpu.VMEM((NW * 8,), jnp.int32),
            pltpu.VMEM((256,), jnp.int32),
            pltpu.VMEM((256, 2 * HID), jnp.float32),
            pltpu.VMEM((NPT, 2 * HID), jnp.float32),
        ],
    )
    return f(msg, dstp, counts)


def _mm(x, w):
    return jnp.dot(x, w, preferred_element_type=jnp.float32)


def _tc_dense0_body(nin, cat, bat, wall,
                    wi1, bi1, wi2, bi2, ww1, bw1, ww2, bw2,
                    wg1, bg1, wg2, bg2, emt, wem, bem, wm1, bm1,
                    a_out, b_out, cond_out):
    x10 = nin[:, :10]
    geo = nin[:, 10:12]
    oh_c = (cat[...] == lax.broadcasted_iota(jnp.int32, (NB, 10), 1)).astype(jnp.float32)
    oh_b = (bat[...] == lax.broadcasted_iota(jnp.int32, (NB, NG), 1)).astype(jnp.float32)
    for b in range(2):
        h0 = jnp.tanh(_mm(jnp.tanh(_mm(x10, wi1[b]) + bi1[b]), wi2[b]) + bi2[b])
        ef = jnp.tanh(_mm(jnp.tanh(_mm(oh_c, emt[b])), wem[b]) + bem[b])
        wf_tab = _mm(jnp.tanh(_mm(wall[...], ww1[b]) + bw1[b]), ww2[b]) + bw2[b]
        wf = jnp.tanh(_mm(oh_b, wf_tab))
        gf = jnp.tanh(_mm(jnp.tanh(_mm(geo, wg1[b]) + bg1[b]), wg2[b]) + bg2[b])
        cond = jnp.concatenate([ef, wf, gf], axis=1)
        h = jnp.concatenate([h0, cond], axis=1)
        wa = wm1[b, :160, :] - wm1[b, 160:, :]
        a_out[:, b * HID:(b + 1) * HID] = _mm(h, wa) + bm1[b]
        b_out[:, b * HID:(b + 1) * HID] = _mm(h, wm1[b, 160:, :])
        cond_out[b] = cond


def _tc_dense0(nin, cat, bat, wall, p):
    grid = (N // NB,)
    bs_w = lambda shape: pl.BlockSpec((2,) + shape, lambda i: (0, 0, 0))
    f = pl.pallas_call(
        _tc_dense0_body,
        grid=grid,
        in_specs=[
            pl.BlockSpec((NB, 12), lambda i: (i, 0)),
            pl.BlockSpec((NB, 1), lambda i: (i, 0)),
            pl.BlockSpec((NB, 1), lambda i: (i, 0)),
            pl.BlockSpec((NG, 1), lambda i: (0, 0)),
            bs_w((10, HID)), bs_w((1, HID)), bs_w((HID, HID)), bs_w((1, HID)),
            bs_w((1, EMB)), bs_w((1, EMB)), bs_w((EMB, EMB)), bs_w((1, EMB)),
            bs_w((2, EMB)), bs_w((1, EMB)), bs_w((EMB, EMB)), bs_w((1, EMB)),
            bs_w((10, EMB)), bs_w((EMB, EMB)), bs_w((1, EMB)),
            bs_w((2 * (HID + COND), HID)), bs_w((1, HID)),
        ],
        out_specs=[
            pl.BlockSpec((NB, 2 * HID), lambda i: (i, 0)),
            pl.BlockSpec((NB, 2 * HID), lambda i: (i, 0)),
            pl.BlockSpec((2, NB, COND), lambda i: (0, i, 0)),
        ],
        out_shape=[
            jax.ShapeDtypeStruct((N, 2 * HID), jnp.float32),
            jax.ShapeDtypeStruct((N, 2 * HID), jnp.float32),
            jax.ShapeDtypeStruct((2, N, COND), jnp.float32),
        ],
    )
    return f(nin, cat, bat, wall,
             p["wi1"], p["bi1"], p["wi2"], p["bi2"],
             p["ww1"], p["bw1"], p["ww2"], p["bw2"],
             p["wg1"], p["bg1"], p["wg2"], p["bg2"],
             p["emt"], p["wem"], p["bem"], p["wm1"], p["bm1"])


def _tc_msg_body(ag, bg, w2, b2, out):
    z = jnp.tanh(ag[...] + bg[...])
    out[...] = _mm(z, w2[...]) + b2[...]


def _tc_msg(ag, bg, w2d, b2c):
    blk = 1024
    grid = (EPN // blk,)
    f = pl.pallas_call(
        _tc_msg_body,
        grid=grid,
        in_specs=[
            pl.BlockSpec((blk, 2 * HID), lambda i: (i, 0)),
            pl.BlockSpec((blk, 2 * HID), lambda i: (i, 0)),
            pl.BlockSpec((2 * HID, 2 * HID), lambda i: (0, 0)),
            pl.BlockSpec((1, 2 * HID), lambda i: (0, 0)),
        ],
        out_specs=pl.BlockSpec((blk, 2 * HID), lambda i: (i, 0)),
        out_shape=jax.ShapeDtypeStruct((EPN, 2 * HID), jnp.float32),
    )
    return f(ag, bg, w2d, b2c)


def _tc_mid_body(agg, cond, wm, bm, a_out, b_out):
    for b in range(2):
        a = agg[:, b * HID:(b + 1) * HID]
        h1 = jnp.tanh(jnp.where(a == NEG_INF, 0.0, a))
        h = jnp.concatenate([h1, cond[b]], axis=1)
        wa = wm[b, :160, :] - wm[b, 160:, :]
        a_out[:, b * HID:(b + 1) * HID] = _mm(h, wa) + bm[b]
        b_out[:, b * HID:(b + 1) * HID] = _mm(h, wm[b, 160:, :])


def _tc_mid(agg, cond, wm, bm):
    grid = (N // NB,)
    f = pl.pallas_call(
        _tc_mid_body,
        grid=grid,
        in_specs=[
            pl.BlockSpec((NB, 2 * HID), lambda i: (i, 0)),
            pl.BlockSpec((2, NB, COND), lambda i: (0, i, 0)),
            pl.BlockSpec((2, 2 * (HID + COND), HID), lambda i: (0, 0, 0)),
            pl.BlockSpec((2, 1, HID), lambda i: (0, 0, 0)),
        ],
        out_specs=[
            pl.BlockSpec((NB, 2 * HID), lambda i: (i, 0)),
            pl.BlockSpec((NB, 2 * HID), lambda i: (i, 0)),
        ],
        out_shape=[
            jax.ShapeDtypeStruct((N, 2 * HID), jnp.float32),
            jax.ShapeDtypeStruct((N, 2 * HID), jnp.float32),
        ],
    )
    return f(agg, cond, wm, bm)


def _tc_tail_body(agg, cond, wt1, bt1, wt2, bt2, out):
    for b in range(2):
        a = agg[:, b * HID:(b + 1) * HID]
        h2 = jnp.tanh(jnp.where(a == NEG_INF, 0.0, a))
        h = jnp.concatenate([h2, cond[b]], axis=1)
        out[b] = _mm(jnp.tanh(_mm(h, wt1[b]) + bt1[b]), wt2[b]) + bt2[b]


def _tc_tail(agg, cond, wt1, bt1, wt2, bt2):
    grid = (N // NB,)
    f = pl.pallas_call(
        _tc_tail_body,
        grid=grid,
        in_specs=[
            pl.BlockSpec((NB, 2 * HID), lambda i: (i, 0)),
            pl.BlockSpec((2, NB, COND), lambda i: (0, i, 0)),
            pl.BlockSpec((2, HID + COND, HID), lambda i: (0, 0, 0)),
            pl.BlockSpec((2, 1, HID), lambda i: (0, 0, 0)),
            pl.BlockSpec((2, HID, 8), lambda i: (0, 0, 0)),
            pl.BlockSpec((2, 1, 8), lambda i: (0, 0, 0)),
        ],
        out_specs=pl.BlockSpec((2, NB, 8), lambda i: (0, i, 0)),
        out_shape=jax.ShapeDtypeStruct((2, N, 8), jnp.float32),
    )
    return f(agg, cond, wt1, bt1, wt2, bt2)


def _stack_params(params):
    q = [params["q1"], params["q2"]]

    def st(path):
        def get(p):
            v = p
            for k in path:
                v = v[k]
            return v
        return jnp.stack([get(q[0]), get(q[1])])

    def lin2(v, r=None):
        return v[:, None, :]

    p = {
        "wi1": st(["init_enc", "l1", "W"]), "bi1": lin2(st(["init_enc", "l1", "b"])),
        "wi2": st(["init_enc", "l2", "W"]), "bi2": lin2(st(["init_enc", "l2", "b"])),
        "ww1": st(["wall_enc", "l1", "W"]), "bw1": lin2(st(["wall_enc", "l1", "b"])),
        "ww2": st(["wall_enc", "l2", "W"]), "bw2": lin2(st(["wall_enc", "l2", "b"])),
        "wg1": st(["geo_enc", "l1", "W"]), "bg1": lin2(st(["geo_enc", "l1", "b"])),
        "wg2": st(["geo_enc", "l2", "W"]), "bg2": lin2(st(["geo_enc", "l2", "b"])),
        "emt": st(["emb_table"]),
        "wem": st(["emb_lin", "W"]), "bem": lin2(st(["emb_lin", "b"])),
        "wm1": st(["mlp1", "l1", "W"]), "bm1": lin2(st(["mlp1", "l1", "b"])),
        "wm1b": st(["mlp1", "l2", "W"]), "bm1b": lin2(st(["mlp1", "l2", "b"])),
        "wm2": st(["mlp2", "l1", "W"]), "bm2": lin2(st(["mlp2", "l1", "b"])),
        "wm2b": st(["mlp2", "l2", "W"]), "bm2b": lin2(st(["mlp2", "l2", "b"])),
        "wt1": st(["tail", "l1", "W"]), "bt1": lin2(st(["tail", "l1", "b"])),
    }
    for i, nm in ((1, "mlp1"), (2, "mlp2")):
        w = st([nm, "l2", "W"])
        bb = st([nm, "l2", "b"])
        wd = jnp.zeros((2 * HID, 2 * HID), jnp.float32)
        wd = wd.at[:HID, :HID].set(w[0]).at[HID:, HID:].set(w[1])
        p[f"w2d{i}"] = wd
        p[f"b2c{i}"] = jnp.concatenate([bb[0], bb[1]])[None, :]

    wt2 = st(["tail", "l2", "W"])
    bt2 = lin2(st(["tail", "l2", "b"]))
    p["wt2"] = jnp.pad(wt2, ((0, 0), (0, 0), (0, 7)))
    p["bt2"] = jnp.pad(bt2, ((0, 0), (0, 0), (0, 7)))
    return p


def kernel(x, actions, tar_scores, geo, wall, category, batch, edge_index, params):
    cat = category.astype(jnp.int32)[:, None]
    bat = batch.astype(jnp.int32)[:, None]
    ei = edge_index.astype(jnp.int32)
    nin = jnp.concatenate([x, actions, tar_scores, geo], axis=1)
    p = _stack_params(params)

    srcp, dstp, counts = _sc_partition(ei[0], ei[1])

    a1, b1, cond = _tc_dense0(nin, cat, bat, wall, p)
    ag, bg = _sc_gather(a1, b1, srcp, dstp, counts)
    m1 = _tc_msg(ag, bg, p["w2d1"], p["b2c1"])
    agg1 = _sc_scatter(m1, dstp, counts)

    a2, b2 = _tc_mid(agg1[:N, :], cond, p["wm2"], p["bm2"])
    ag2, bg2 = _sc_gather(a2, b2, srcp, dstp, counts)
    m2 = _tc_msg(ag2, bg2, p["w2d2"], p["b2c2"])
    agg2 = _sc_scatter(m2, dstp, counts)

    q = _tc_tail(agg2[:N, :], cond, p["wt1"], p["bt1"], p["wt2"], p["bt2"])
    return (q[0, :, :1], q[1, :, :1])


# speedup vs baseline: 3.6254x; 1.0789x over previous
import functools

import jax
import jax.numpy as jnp
from jax import lax
from jax.experimental import pallas as pl
from jax.experimental.pallas import tpu as pltpu
from jax.experimental.pallas import tpu_sc as plsc

N = 10000
E = 320000
NG = 256
HID = 64
EMB = 32
COND = 3 * EMB
NW = 32
NPT = 320
MAGIC = 13108
EPN = 328704
NB = 1000
EROW = E + 8
STAG = 5408
TRASH = 5392
PCH = 1280
NPCH = E // PCH
NEG_INF = float("-inf")


def _wid():
    return lax.axis_index("s") * 2 + lax.axis_index("c")


def _iota16():
    return lax.broadcasted_iota(jnp.int32, (16,), 0)


def _scal(v):
    return jnp.sum(jnp.where(_iota16() == 0, v, 0))


def _lane(v16, j):
    jv = jnp.full((16,), j, jnp.int32)
    g = jax.lax.gather(
        v16, jv[:, None],
        jax.lax.GatherDimensionNumbers(offset_dims=(), collapsed_slice_dims=(0,),
                                       start_index_map=(0,)),
        (1,), mode=jax.lax.GatherScatterMode.PROMISE_IN_BOUNDS)
    return g[0]


def _tile_cnt_off(cvm, t):
    cnt = jnp.int32(0)
    off = jnp.int32(0)
    for g in range(2):
        gt = _iota16() + 16 * g
        cg = plsc.load_gather(cvm, [gt * 8])
        r256 = jnp.bitwise_and(cg + 255, jnp.int32(-256))
        off = off + jnp.sum(jnp.where(gt < t, r256, 0))
        cnt = cnt + jnp.sum(jnp.where(gt == t, cg, 0))
    return cnt, off


def _sc_partition_body(src_e, dst_e, srcp, dstp, counts,
                       sbuf, dbuf, stag, dtag, cbuf, ss0, ss1, sd0, sd1):
    t = _wid()
    zero16 = jnp.zeros((16,), jnp.int32)
    sems = ((ss0, sd0), (ss1, sd1))

    def zinit(i, _):
        stag[pl.ds(i * 16, 16)] = zero16
        dtag[pl.ds(i * 16, 16)] = zero16
        return 0

    lax.fori_loop(0, STAG // 16, zinit, 0)

    def issue(ch, p):
        cs = pltpu.async_copy(src_e.at[pl.ds(pl.multiple_of(ch * PCH, 8), PCH)],
                              sbuf.at[p], sems[p][0])
        cd = pltpu.async_copy(dst_e.at[pl.ds(pl.multiple_of(ch * PCH, 8), PCH)],
                              dbuf.at[p], sems[p][1])
        return cs, cd

    pre = issue(0, 0)

    def pair(i, carry):
        cursor, written = carry
        for p in range(2):
            ch = 2 * i + p

            @pl.when(ch + 1 < NPCH)
            def _():
                issue(ch + 1, 1 - p)

            pltpu.make_async_copy(src_e.at[pl.ds(0, PCH)], sbuf.at[p], sems[p][0]).wait()
            pltpu.make_async_copy(dst_e.at[pl.ds(0, PCH)], dbuf.at[p], sems[p][1]).wait()
            curv = jnp.full((16,), cursor, jnp.int32)
            for v in range(PCH // 16):
                s16 = sbuf[p, pl.ds(v * 16, 16)]
                d16 = dbuf[p, pl.ds(v * 16, 16)]
                own = jnp.right_shift(d16 * MAGIC, 22)
                m = own == t
                mi = m.astype(jnp.int32)
                rank = plsc.cumsum(mi) - mi
                dest = jnp.where(m, curv + rank, TRASH + _iota16())
                plsc.store_scatter(stag, [dest], s16)
                plsc.store_scatter(dtag, [dest], d16)
                curv = curv + plsc.all_reduce_population_count(m)
            cursor = _scal(curv)

            def flush(cur, wr):
                pltpu.sync_copy(stag.at[pl.ds(0, 4096)],
                                srcp.at[pl.ds(pl.multiple_of(t * EROW + wr, 8), 4096)])
                pltpu.sync_copy(dtag.at[pl.ds(0, 4096)],
                                dstp.at[pl.ds(pl.multiple_of(t * EROW + wr, 8), 4096)])
                for v in range(82):
                    stag[pl.ds(v * 16, 16)] = stag[pl.ds(4096 + v * 16, 16)]
                    dtag[pl.ds(v * 16, 16)] = dtag[pl.ds(4096 + v * 16, 16)]
                return cur - 4096, wr + 4096

            cursor, written = lax.cond(cursor >= 4096, flush,
                                       lambda c, w: (c, w), cursor, written)
        return cursor, written

    cursor, written = lax.fori_loop(0, NPCH // 2, pair,
                                    (jnp.int32(0), jnp.int32(0)))

    def tail(i, _):
        pltpu.sync_copy(stag.at[pl.ds(i * 8, 8)],
                        srcp.at[pl.ds(pl.multiple_of(t * EROW + written + i * 8, 8), 8)])
        pltpu.sync_copy(dtag.at[pl.ds(i * 8, 8)],
                        dstp.at[pl.ds(pl.multiple_of(t * EROW + written + i * 8, 8), 8)])
        return 0

    lax.fori_loop(0, jnp.right_shift(cursor + 7, 3), tail, 0)
    cbuf[...] = jnp.full((16,), written + cursor, jnp.int32)
    pltpu.sync_copy(cbuf.at[pl.ds(0, 8)], counts.at[pl.ds(pl.multiple_of(t * 8, 8), 8)])


def _sc_partition(src_e, dst_e):
    mesh = plsc.VectorSubcoreMesh(core_axis_name="c", subcore_axis_name="s")
    f = pl.kernel(
        _sc_partition_body,
        out_type=[
            jax.ShapeDtypeStruct((NW * EROW,), jnp.int32),
            jax.ShapeDtypeStruct((NW * EROW,), jnp.int32),
            jax.ShapeDtypeStruct((NW * 8,), jnp.int32),
        ],
        mesh=mesh,
        compiler_params=pltpu.CompilerParams(needs_layout_passes=False),
        scratch_types=[
            pltpu.VMEM((2, PCH), jnp.int32),
            pltpu.VMEM((2, PCH), jnp.int32),
            pltpu.VMEM((STAG,), jnp.int32),
            pltpu.VMEM((STAG,), jnp.int32),
            pltpu.VMEM((16,), jnp.int32),
            pltpu.SemaphoreType.DMA,
            pltpu.SemaphoreType.DMA,
            pltpu.SemaphoreType.DMA,
            pltpu.SemaphoreType.DMA,
        ],
    )
    return f(src_e, dst_e)


def _sc_gather_body(atab, btab, srcp, dstp, counts, ag, bg,
                    cvm, sloc, dloc, ia, ia2, ib, ib2, abuf, bbuf,
                    sem_a, sem_a2, sem_b, sem_b2):
    t = _wid()
    pltpu.sync_copy(counts, cvm)
    cnt, off = _tile_cnt_off(cvm, t)
    nch = jnp.right_shift(cnt + 255, 8)

    def chunk(c, _):
        pltpu.sync_copy(srcp.at[pl.ds(pl.multiple_of(t * EROW + c * 256, 8), 256)], sloc)
        pltpu.sync_copy(dstp.at[pl.ds(pl.multiple_of(t * EROW + c * 256, 8), 256)], dloc)
        for v in range(16):
            lane = c * 256 + v * 16 + _iota16()
            ok = lane < cnt
            dv = jnp.where(ok, dloc[pl.ds(v * 16, 16)], 0)
            sv = jnp.where(ok, sloc[pl.ds(v * 16, 16)], 0)
            if v < 8:
                ia[pl.ds(v * 16, 16)] = dv
                ib[pl.ds(v * 16, 16)] = sv
            else:
                ia2[pl.ds((v - 8) * 16, 16)] = dv
                ib2[pl.ds((v - 8) * 16, 16)] = sv
        c1 = pltpu.async_copy(atab.at[ia], abuf.at[pl.ds(0, 128), :], sem_a)
        c2 = pltpu.async_copy(atab.at[ia2], abuf.at[pl.ds(128, 128), :], sem_a2)
        c3 = pltpu.async_copy(btab.at[ib], bbuf.at[pl.ds(0, 128), :], sem_b)
        c4 = pltpu.async_copy(btab.at[ib2], bbuf.at[pl.ds(128, 128), :], sem_b2)
        c1.wait()
        c2.wait()
        c3.wait()
        c4.wait()
        pltpu.sync_copy(abuf, ag.at[pl.ds(pl.multiple_of(off + c * 256, 8), 256), :])
        pltpu.sync_copy(bbuf, bg.at[pl.ds(pl.multiple_of(off + c * 256, 8), 256), :])
        return 0

    lax.fori_loop(0, nch, chunk, 0)


def _sc_gather(atab, btab, srcp, dstp, counts):
    mesh = plsc.VectorSubcoreMesh(core_axis_name="c", subcore_axis_name="s")
    f = pl.kernel(
        _sc_gather_body,
        out_type=[
            jax.ShapeDtypeStruct((EPN, 2 * HID), jnp.float32),
            jax.ShapeDtypeStruct((EPN, 2 * HID), jnp.float32),
        ],
        mesh=mesh,
        compiler_params=pltpu.CompilerParams(needs_layout_passes=False),
        scratch_types=[
            pltpu.VMEM((NW * 8,), jnp.int32),
            pltpu.VMEM((256,), jnp.int32),
            pltpu.VMEM((256,), jnp.int32),
            pltpu.VMEM((128,), jnp.int32),
            pltpu.VMEM((128,), jnp.int32),
            pltpu.VMEM((128,), jnp.int32),
            pltpu.VMEM((128,), jnp.int32),
            pltpu.VMEM((256, 2 * HID), jnp.float32),
            pltpu.VMEM((256, 2 * HID), jnp.float32),
            pltpu.SemaphoreType.DMA,
            pltpu.SemaphoreType.DMA,
            pltpu.SemaphoreType.DMA,
            pltpu.SemaphoreType.DMA,
        ],
    )
    return f(atab, btab, srcp, dstp, counts)


def _sc_scatter_body(msg, dstp, counts, agg, cvm, dbuf, mbuf, acc, sem_m0, sem_m1):
    t = _wid()
    pltpu.sync_copy(counts, cvm)
    cnt, off = _tile_cnt_off(cvm, t)
    base = t * NPT
    ninf = jnp.full((16,), NEG_INF, jnp.float32)

    def zinit(r, _):
        for v in range(8):
            acc[r, pl.ds(v * 16, 16)] = ninf
        return 0

    lax.fori_loop(0, NPT, zinit, 0)

    nch = jnp.right_shift(cnt + 255, 8)
    msems = (sem_m0, sem_m1)

    def mload(c, p):
        pltpu.async_copy(msg.at[pl.ds(pl.multiple_of(off + c * 256, 8), 256), :],
                         mbuf.at[p], msems[p])

    @pl.when(nch > 0)
    def _():
        mload(0, 0)

    def pairchunk(i, _):
        for p in range(2):
            c = 2 * i + p

            @pl.when(c < nch)
            def _():
                pltpu.make_async_copy(
                    msg.at[pl.ds(0, 256), :], mbuf.at[p], msems[p]).wait()

                @pl.when(c + 1 < nch)
                def _():
                    mload(c + 1, 1 - p)

                pltpu.sync_copy(
                    dstp.at[pl.ds(pl.multiple_of(t * EROW + c * 256, 8), 256)], dbuf)
                n_c = cnt - c * 256
                nv = jnp.minimum(n_c, jnp.int32(256))
                nfull = jnp.right_shift(nv, 4)

                def vreg16(v, _):
                    d16 = dbuf[pl.ds(v * 16, 16)] - base
                    for j in range(16):
                        d = _lane(d16, j)
                        e = v * 16 + j
                        for f in range(8):
                            mv = mbuf[p, e, pl.ds(f * 16, 16)]
                            av = acc[d, pl.ds(f * 16, 16)]
                            acc[d, pl.ds(f * 16, 16)] = jnp.maximum(av, mv)
                    return 0

                lax.fori_loop(0, nfull, vreg16, 0)
                rem = jnp.bitwise_and(nv, 15)

                @pl.when(rem > 0)
                def _():
                    v = nfull
                    d16 = dbuf[pl.ds(v * 16, 16)] - base
                    for j in range(16):
                        d = _lane(d16, j)
                        e = v * 16 + j

                        @pl.when(j < rem)
                        def _():
                            for f in range(8):
                                mv = mbuf[p, e, pl.ds(f * 16, 16)]
                                av = acc[d, pl.ds(f * 16, 16)]
                                acc[d, pl.ds(f * 16, 16)] = jnp.maximum(av, mv)
        return 0

    npair = jnp.right_shift(nch + 1, 1)
    lax.fori_loop(0, npair, pairchunk, 0)
    pltpu.sync_copy(acc, agg.at[pl.ds(base, NPT), :])


def _sc_scatter(msg, dstp, counts):
    mesh = plsc.VectorSubcoreMesh(core_axis_name="c", subcore_axis_name="s")
    f = pl.kernel(
        _sc_scatter_body,
        out_type=jax.ShapeDtypeStruct((NW * NPT, 2 * HID), jnp.float32),
        mesh=mesh,
        compiler_params=pltpu.CompilerParams(needs_layout_passes=False),
        scratch_types=[
            pltpu.VMEM((NW * 8,), jnp.int32),
            pltpu.VMEM((256,), jnp.int32),
            pltpu.VMEM((2, 256, 2 * HID), jnp.float32),
            pltpu.VMEM((NPT, 2 * HID), jnp.float32),
            pltpu.SemaphoreType.DMA,
            pltpu.SemaphoreType.DMA,
        ],
    )
    return f(msg, dstp, counts)


def _mm(x, w):
    return jnp.dot(x, w, preferred_element_type=jnp.float32)


def _tc_dense0_body(nin, cat, bat, wall,
                    wi1, bi1, wi2, bi2, ww1, bw1, ww2, bw2,
                    wg1, bg1, wg2, bg2, emt, wem, bem, wm1, bm1,
                    a_out, b_out, cond_out):
    x10 = nin[:, :10]
    geo = nin[:, 10:12]
    oh_c = (cat[...] == lax.broadcasted_iota(jnp.int32, (NB, 10), 1)).astype(jnp.float32)
    oh_b = (bat[...] == lax.broadcasted_iota(jnp.int32, (NB, NG), 1)).astype(jnp.float32)
    for b in range(2):
        h0 = jnp.tanh(_mm(jnp.tanh(_mm(x10, wi1[b]) + bi1[b]), wi2[b]) + bi2[b])
        ef = jnp.tanh(_mm(jnp.tanh(_mm(oh_c, emt[b])), wem[b]) + bem[b])
        wf_tab = _mm(jnp.tanh(_mm(wall[...], ww1[b]) + bw1[b]), ww2[b]) + bw2[b]
        wf = jnp.tanh(_mm(oh_b, wf_tab))
        gf = jnp.tanh(_mm(jnp.tanh(_mm(geo, wg1[b]) + bg1[b]), wg2[b]) + bg2[b])
        cond = jnp.concatenate([ef, wf, gf], axis=1)
        h = jnp.concatenate([h0, cond], axis=1)
        wa = wm1[b, :160, :] - wm1[b, 160:, :]
        a_out[:, b * HID:(b + 1) * HID] = _mm(h, wa) + bm1[b]
        b_out[:, b * HID:(b + 1) * HID] = _mm(h, wm1[b, 160:, :])
        cond_out[b] = cond


def _tc_dense0(nin, cat, bat, wall, p):
    grid = (N // NB,)
    bs_w = lambda shape: pl.BlockSpec((2,) + shape, lambda i: (0, 0, 0))
    f = pl.pallas_call(
        _tc_dense0_body,
        grid=grid,
        in_specs=[
            pl.BlockSpec((NB, 12), lambda i: (i, 0)),
            pl.BlockSpec((NB, 1), lambda i: (i, 0)),
            pl.BlockSpec((NB, 1), lambda i: (i, 0)),
            pl.BlockSpec((NG, 1), lambda i: (0, 0)),
            bs_w((10, HID)), bs_w((1, HID)), bs_w((HID, HID)), bs_w((1, HID)),
            bs_w((1, EMB)), bs_w((1, EMB)), bs_w((EMB, EMB)), bs_w((1, EMB)),
            bs_w((2, EMB)), bs_w((1, EMB)), bs_w((EMB, EMB)), bs_w((1, EMB)),
            bs_w((10, EMB)), bs_w((EMB, EMB)), bs_w((1, EMB)),
            bs_w((2 * (HID + COND), HID)), bs_w((1, HID)),
        ],
        out_specs=[
            pl.BlockSpec((NB, 2 * HID), lambda i: (i, 0)),
            pl.BlockSpec((NB, 2 * HID), lambda i: (i, 0)),
            pl.BlockSpec((2, NB, COND), lambda i: (0, i, 0)),
        ],
        out_shape=[
            jax.ShapeDtypeStruct((N, 2 * HID), jnp.float32),
            jax.ShapeDtypeStruct((N, 2 * HID), jnp.float32),
            jax.ShapeDtypeStruct((2, N, COND), jnp.float32),
        ],
    )
    return f(nin, cat, bat, wall,
             p["wi1"], p["bi1"], p["wi2"], p["bi2"],
             p["ww1"], p["bw1"], p["ww2"], p["bw2"],
             p["wg1"], p["bg1"], p["wg2"], p["bg2"],
             p["emt"], p["wem"], p["bem"], p["wm1"], p["bm1"])


def _tc_msg_body(ag, bg, w2, b2, out):
    z = jnp.tanh(ag[...] + bg[...])
    out[...] = _mm(z, w2[...]) + b2[...]


def _tc_msg(ag, bg, w2d, b2c):
    blk = 1024
    grid = (EPN // blk,)
    f = pl.pallas_call(
        _tc_msg_body,
        grid=grid,
        in_specs=[
            pl.BlockSpec((blk, 2 * HID), lambda i: (i, 0)),
            pl.BlockSpec((blk, 2 * HID), lambda i: (i, 0)),
            pl.BlockSpec((2 * HID, 2 * HID), lambda i: (0, 0)),
            pl.BlockSpec((1, 2 * HID), lambda i: (0, 0)),
        ],
        out_specs=pl.BlockSpec((blk, 2 * HID), lambda i: (i, 0)),
        out_shape=jax.ShapeDtypeStruct((EPN, 2 * HID), jnp.float32),
    )
    return f(ag, bg, w2d, b2c)


def _tc_mid_body(agg, cond, wm, bm, a_out, b_out):
    for b in range(2):
        a = agg[:, b * HID:(b + 1) * HID]
        h1 = jnp.tanh(jnp.where(a == NEG_INF, 0.0, a))
        h = jnp.concatenate([h1, cond[b]], axis=1)
        wa = wm[b, :160, :] - wm[b, 160:, :]
        a_out[:, b * HID:(b + 1) * HID] = _mm(h, wa) + bm[b]
        b_out[:, b * HID:(b + 1) * HID] = _mm(h, wm[b, 160:, :])


def _tc_mid(agg, cond, wm, bm):
    grid = (N // NB,)
    f = pl.pallas_call(
        _tc_mid_body,
        grid=grid,
        in_specs=[
            pl.BlockSpec((NB, 2 * HID), lambda i: (i, 0)),
            pl.BlockSpec((2, NB, COND), lambda i: (0, i, 0)),
            pl.BlockSpec((2, 2 * (HID + COND), HID), lambda i: (0, 0, 0)),
            pl.BlockSpec((2, 1, HID), lambda i: (0, 0, 0)),
        ],
        out_specs=[
            pl.BlockSpec((NB, 2 * HID), lambda i: (i, 0)),
            pl.BlockSpec((NB, 2 * HID), lambda i: (i, 0)),
        ],
        out_shape=[
            jax.ShapeDtypeStruct((N, 2 * HID), jnp.float32),
            jax.ShapeDtypeStruct((N, 2 * HID), jnp.float32),
        ],
    )
    return f(agg, cond, wm, bm)


def _tc_tail_body(agg, cond, wt1, bt1, wt2, bt2, out):
    for b in range(2):
        a = agg[:, b * HID:(b + 1) * HID]
        h2 = jnp.tanh(jnp.where(a == NEG_INF, 0.0, a))
        h = jnp.concatenate([h2, cond[b]], axis=1)
        out[b] = _mm(jnp.tanh(_mm(h, wt1[b]) + bt1[b]), wt2[b]) + bt2[b]


def _tc_tail(agg, cond, wt1, bt1, wt2, bt2):
    grid = (N // NB,)
    f = pl.pallas_call(
        _tc_tail_body,
        grid=grid,
        in_specs=[
            pl.BlockSpec((NB, 2 * HID), lambda i: (i, 0)),
            pl.BlockSpec((2, NB, COND), lambda i: (0, i, 0)),
            pl.BlockSpec((2, HID + COND, HID), lambda i: (0, 0, 0)),
            pl.BlockSpec((2, 1, HID), lambda i: (0, 0, 0)),
            pl.BlockSpec((2, HID, 8), lambda i: (0, 0, 0)),
            pl.BlockSpec((2, 1, 8), lambda i: (0, 0, 0)),
        ],
        out_specs=pl.BlockSpec((2, NB, 8), lambda i: (0, i, 0)),
        out_shape=jax.ShapeDtypeStruct((2, N, 8), jnp.float32),
    )
    return f(agg, cond, wt1, bt1, wt2, bt2)


def _stack_params(params):
    q = [params["q1"], params["q2"]]

    def st(path):
        def get(p):
            v = p
            for k in path:
                v = v[k]
            return v
        return jnp.stack([get(q[0]), get(q[1])])

    def lin2(v, r=None):
        return v[:, None, :]

    p = {
        "wi1": st(["init_enc", "l1", "W"]), "bi1": lin2(st(["init_enc", "l1", "b"])),
        "wi2": st(["init_enc", "l2", "W"]), "bi2": lin2(st(["init_enc", "l2", "b"])),
        "ww1": st(["wall_enc", "l1", "W"]), "bw1": lin2(st(["wall_enc", "l1", "b"])),
        "ww2": st(["wall_enc", "l2", "W"]), "bw2": lin2(st(["wall_enc", "l2", "b"])),
        "wg1": st(["geo_enc", "l1", "W"]), "bg1": lin2(st(["geo_enc", "l1", "b"])),
        "wg2": st(["geo_enc", "l2", "W"]), "bg2": lin2(st(["geo_enc", "l2", "b"])),
        "emt": st(["emb_table"]),
        "wem": st(["emb_lin", "W"]), "bem": lin2(st(["emb_lin", "b"])),
        "wm1": st(["mlp1", "l1", "W"]), "bm1": lin2(st(["mlp1", "l1", "b"])),
        "wm1b": st(["mlp1", "l2", "W"]), "bm1b": lin2(st(["mlp1", "l2", "b"])),
        "wm2": st(["mlp2", "l1", "W"]), "bm2": lin2(st(["mlp2", "l1", "b"])),
        "wm2b": st(["mlp2", "l2", "W"]), "bm2b": lin2(st(["mlp2", "l2", "b"])),
        "wt1": st(["tail", "l1", "W"]), "bt1": lin2(st(["tail", "l1", "b"])),
    }
    for i, nm in ((1, "mlp1"), (2, "mlp2")):
        w = st([nm, "l2", "W"])
        bb = st([nm, "l2", "b"])
        wd = jnp.zeros((2 * HID, 2 * HID), jnp.float32)
        wd = wd.at[:HID, :HID].set(w[0]).at[HID:, HID:].set(w[1])
        p[f"w2d{i}"] = wd
        p[f"b2c{i}"] = jnp.concatenate([bb[0], bb[1]])[None, :]

    wt2 = st(["tail", "l2", "W"])
    bt2 = lin2(st(["tail", "l2", "b"]))
    p["wt2"] = jnp.pad(wt2, ((0, 0), (0, 0), (0, 7)))
    p["bt2"] = jnp.pad(bt2, ((0, 0), (0, 0), (0, 7)))
    return p


def kernel(x, actions, tar_scores, geo, wall, category, batch, edge_index, params):
    cat = category.astype(jnp.int32)[:, None]
    bat = batch.astype(jnp.int32)[:, None]
    ei = edge_index.astype(jnp.int32)
    nin = jnp.concatenate([x, actions, tar_scores, geo], axis=1)
    p = _stack_params(params)

    srcp, dstp, counts = _sc_partition(ei[0], ei[1])

    a1, b1, cond = _tc_dense0(nin, cat, bat, wall, p)
    ag, bg = _sc_gather(a1, b1, srcp, dstp, counts)
    m1 = _tc_msg(ag, bg, p["w2d1"], p["b2c1"])
    agg1 = _sc_scatter(m1, dstp, counts)

    a2, b2 = _tc_mid(agg1[:N, :], cond, p["wm2"], p["bm2"])
    ag2, bg2 = _sc_gather(a2, b2, srcp, dstp, counts)
    m2 = _tc_msg(ag2, bg2, p["w2d2"], p["b2c2"])
    agg2 = _sc_scatter(m2, dstp, counts)

    q = _tc_tail(agg2[:N, :], cond, p["wt1"], p["bt1"], p["wt2"], p["bt2"])
    return (q[0, :, :1], q[1, :, :1])
